# baseline (speedup 1.0000x reference)
import jax
import jax.numpy as jnp
import numpy as np
from jax.experimental import pallas as pl

S = 2048
DIM = 2048
H = 16
D = 128
ROPE = 64
QLORA = 1536
TOPK = 1024
SOFTMAX_SCALE = D ** (-0.5)

BS = 256
BT = 512


def _hadamard(n):
    m = np.array([[1.0]], dtype=np.float32)
    while m.shape[0] < n:
        m = np.block([[m, m], [m, -m]]).astype(np.float32)
    return m

_HAD_NP = _hadamard(D)


def _row_mean(t):
    acc = t[:, 0:8]
    for i in range(1, 16):
        acc = acc + t[:, 8 * i:8 * i + 8]
    while acc.shape[1] > 1:
        h = acc.shape[1] // 2
        acc = acc[:, :h] + acc[:, h:]
    return acc * (1.0 / 128.0)


def _prep_kernel(x_ref, qr_ref, cos_ref, sin_ref, wq_ref, wk_ref, wp_ref,
                 lnw_ref, lnb_ref, had_ref, qf_ref, kdeq_ref, w0_ref):
    xf = x_ref[0].astype(jnp.float32)
    qrf = qr_ref[0].astype(jnp.float32)
    cosb = cos_ref[...]
    sinb = sin_ref[...]
    had = had_ref[...]

    k = jnp.dot(xf, wk_ref[...], preferred_element_type=jnp.float32)
    mu = _row_mean(k)
    var = _row_mean((k - mu) ** 2)
    k = (k - mu) / jnp.sqrt(var + 1e-6) * lnw_ref[...] + lnb_ref[...]
    k_pe = k[:, :ROPE]
    k_nope = k[:, ROPE:]
    k_rot = jnp.concatenate([-k_pe[:, ROPE // 2:], k_pe[:, :ROPE // 2]], axis=1)
    k_pe = k_pe * cosb + k_rot * sinb
    kf = jnp.dot(jnp.concatenate([k_pe, k_nope], axis=1), had,
                 preferred_element_type=jnp.float32) * (D ** -0.5)
    scale = jnp.max(jnp.abs(kf), axis=-1, keepdims=True) / 448.0 + 1e-12
    kdeq_ref[...] = (kf / scale) * scale

    w0_ref[...] = jnp.dot(xf, wp_ref[...],
                          preferred_element_type=jnp.float32) * (H ** -0.5)

    for h in range(H):
        qh = jnp.dot(qrf, wq_ref[h], preferred_element_type=jnp.float32)
        q_pe = qh[:, :ROPE]
        q_nope = qh[:, ROPE:]
        q_rot = jnp.concatenate([-q_pe[:, ROPE // 2:], q_pe[:, :ROPE // 2]],
                                axis=1)
        q_pe = q_pe * cosb + q_rot * sinb
        qf_ref[h] = jnp.dot(jnp.concatenate([q_pe, q_nope], axis=1), had,
                            preferred_element_type=jnp.float32) * (D ** -0.5)


def _score_kernel(qf_ref, kdeq_ref, w0_ref, out_ref):
    i = pl.program_id(0)
    j = pl.program_id(1)
    w0b = w0_ref[...].astype(jnp.bfloat16).astype(jnp.float32)
    kd = kdeq_ref[...]
    terms = []
    for h in range(H):
        logit = jax.lax.dot_general(
            qf_ref[h], kd, (((1,), (1,)), ((), ())),
            preferred_element_type=jnp.float32) * SOFTMAX_SCALE
        lg = jnp.maximum(logit, 0.0).astype(jnp.bfloat16).astype(jnp.float32)
        terms.append(lg * w0b[:, h][:, None])
    while len(terms) > 1:
        terms = [terms[t] + terms[t + 1] for t in range(0, len(terms), 2)]
    acc = terms[0]
    rows = i * BS + jax.lax.broadcasted_iota(jnp.int32, (BS, BT), 0)
    cols = j * BT + jax.lax.broadcasted_iota(jnp.int32, (BS, BT), 1)
    out_ref[...] = jnp.where(cols <= rows, acc, -jnp.inf)


def kernel(x, qr, cos, sin, k_cache, k_scale, Wq, Wk, ln_w, ln_b, Wp):
    del k_cache, k_scale
    had = jnp.asarray(_HAD_NP)
    wq_h = Wq.reshape(QLORA, H, D).transpose(1, 0, 2)

    nb = S // BS
    qf, kdeq, w0 = pl.pallas_call(
        _prep_kernel,
        grid=(nb,),
        in_specs=[
            pl.BlockSpec((1, BS, DIM), lambda i: (0, i, 0)),
            pl.BlockSpec((1, BS, QLORA), lambda i: (0, i, 0)),
            pl.BlockSpec((BS, ROPE), lambda i: (i, 0)),
            pl.BlockSpec((BS, ROPE), lambda i: (i, 0)),
            pl.BlockSpec((H, QLORA, D), lambda i: (0, 0, 0)),
            pl.BlockSpec((DIM, D), lambda i: (0, 0)),
            pl.BlockSpec((DIM, H), lambda i: (0, 0)),
            pl.BlockSpec((D,), lambda i: (0,)),
            pl.BlockSpec((D,), lambda i: (0,)),
            pl.BlockSpec((D, D), lambda i: (0, 0)),
        ],
        out_specs=[
            pl.BlockSpec((H, BS, D), lambda i: (0, i, 0)),
            pl.BlockSpec((BS, D), lambda i: (i, 0)),
            pl.BlockSpec((BS, H), lambda i: (i, 0)),
        ],
        out_shape=[
            jax.ShapeDtypeStruct((H, S, D), jnp.float32),
            jax.ShapeDtypeStruct((S, D), jnp.float32),
            jax.ShapeDtypeStruct((S, H), jnp.float32),
        ],
    )(x, qr, cos, sin, wq_h, Wk, Wp, ln_w, ln_b, had)

    scores = pl.pallas_call(
        _score_kernel,
        grid=(S // BS, S // BT),
        in_specs=[
            pl.BlockSpec((H, BS, D), lambda i, j: (0, i, 0)),
            pl.BlockSpec((BT, D), lambda i, j: (j, 0)),
            pl.BlockSpec((BS, H), lambda i, j: (i, 0)),
        ],
        out_specs=pl.BlockSpec((BS, BT), lambda i, j: (i, j)),
        out_shape=jax.ShapeDtypeStruct((S, S), jnp.float32),
    )(qf, kdeq, w0)

    vals, idx = jax.lax.top_k(scores, TOPK)
    idx = jnp.where(jnp.isinf(vals), -1, idx)
    return vals, idx
